# baseline (speedup 1.0000x reference)
import functools

import jax
import jax.numpy as jnp
from jax import lax
from jax.experimental import pallas as pl
from jax.experimental.pallas import tpu as pltpu
from jax.experimental.pallas import tpu_sc as plsc

_NC = 2
_NS = 16
_L = 16
_NW = _NC * _NS

_CHUNK = 40
_K = 5
_DEG_W = 16


def _sc_aggregate(xh_split, src, dst, n_pad):
    e_total = src.shape[0]
    wr = xh_split.shape[2]
    assert e_total % (_NS * _CHUNK * _K) == 0
    e_per_t = e_total // _NS
    n_groups = e_per_t // (_CHUNK * _K)
    rows_per_tile = n_pad // _NS

    mesh = plsc.VectorSubcoreMesh(
        core_axis_name="c", subcore_axis_name="s",
        num_cores=_NC, num_subcores=_NS)

    @functools.partial(
        pl.kernel,
        out_type=jax.ShapeDtypeStruct((_NC, n_pad, wr), jnp.float32),
        mesh=mesh,
        scratch_types=[
            pltpu.VMEM((2, _K, _CHUNK), jnp.int32),
            pltpu.VMEM((2, _K, _CHUNK), jnp.int32),
            pltpu.VMEM((2, _K, _CHUNK, wr), jnp.float32),
            pltpu.VMEM((_L, wr), jnp.float32),
            pltpu.VMEM_SHARED((n_pad, wr), jnp.float32),
            pltpu.SemaphoreType.DMA,
            pltpu.SemaphoreType.DMA,
            pltpu.SemaphoreType.DMA,
            pltpu.SemaphoreType.DMA,
            pltpu.SemaphoreType.DMA,
            pltpu.SemaphoreType.DMA,
            pltpu.SemaphoreType.DMA,
            pltpu.SemaphoreType.DMA,
        ],
        compiler_params=pltpu.CompilerParams(use_tc_tiling_on_sc=False),
    )
    def agg_kernel(xh_hbm, src_hbm, dst_hbm, acc_out,
                   srcv, dstv, rows, zrow, acc_sh,
                   sem_is0, sem_is1, sem_id0, sem_id1,
                   sem_g0, sem_g1, sem_s0, sem_s1):
        cid = lax.axis_index("c")
        sid = lax.axis_index("s")
        sem_is = (sem_is0, sem_is1)
        sem_id = (sem_id0, sem_id1)
        sem_g = (sem_g0, sem_g1)
        sem_s = (sem_s0, sem_s1)

        for i in range(_L):
            for k in range(wr // _L):
                zrow[i, pl.ds(k * _L, _L)] = jnp.zeros((_L,), jnp.float32)

        zbase = sid * rows_per_tile

        @pl.loop(0, rows_per_tile, step=_L)
        def _(r):
            pltpu.sync_copy(zrow, acc_sh.at[pl.ds(zbase + r, _L)])

        plsc.subcore_barrier()

        ebase = sid * e_per_t

        def issue_src_idx(g, p):
            for b in range(_K):
                off = pl.multiple_of(ebase + (g * _K + b) * _CHUNK, 8)
                pltpu.async_copy(src_hbm.at[pl.ds(off, _CHUNK)],
                                 srcv.at[p, b], sem_is[p])

        def issue_dst_idx(g, p):
            for b in range(_K):
                off = pl.multiple_of(ebase + (g * _K + b) * _CHUNK, 8)
                pltpu.async_copy(dst_hbm.at[pl.ds(off, _CHUNK)],
                                 dstv.at[p, b], sem_id[p])

        def drain_src_idx(p):
            for b in range(_K):
                pltpu.make_async_copy(src_hbm.at[pl.ds(ebase, _CHUNK)],
                                      srcv.at[p, b], sem_is[p]).wait()

        def drain_dst_idx(p):
            for b in range(_K):
                pltpu.make_async_copy(dst_hbm.at[pl.ds(ebase, _CHUNK)],
                                      dstv.at[p, b], sem_id[p]).wait()

        def drain_scatters(p):
            for b in range(_K):
                pltpu.make_async_copy(rows.at[p, b],
                                      acc_sh.at[dstv.at[p, b]],
                                      sem_s[p]).wait()

        issue_src_idx(0, 0)
        issue_src_idx(1, 1)

        @pl.loop(0, n_groups, step=2)
        def _(g):
            for p in range(2):
                @pl.when(g >= 2)
                def _():
                    drain_scatters(p)
                issue_dst_idx(g + p, p)
                drain_src_idx(p)
                gathers = [
                    pltpu.async_copy(xh_hbm.at[cid].at[srcv.at[p, b]],
                                     rows.at[p, b], sem_g[p])
                    for b in range(_K)
                ]

                for d in gathers:
                    d.wait()

                @pl.when(g + p + 2 < n_groups)
                def _():
                    issue_src_idx(g + p + 2, p)
                drain_dst_idx(p)
                for b in range(_K):
                    pltpu.async_copy(rows.at[p, b],
                                     acc_sh.at[dstv.at[p, b]],
                                     sem_s[p], add=True)

        drain_scatters(0)
        drain_scatters(1)

        plsc.subcore_barrier()

        pltpu.sync_copy(acc_sh.at[pl.ds(zbase, rows_per_tile)],
                        acc_out.at[cid, pl.ds(zbase, rows_per_tile)])

    return agg_kernel(xh_split, src, dst)


def _tc_dense(acc, x, h_0, c_0, w_full, bias, lin_w, lin_b,
              d_in, d_out, periods, blk):
    n = x.shape[0]
    assert n % blk == 0
    wx = d_in + d_out

    w2 = wx // 2
    wr = w2 + _DEG_W

    def body(acc_ref, x_ref, h_ref, c_ref, wf_ref, b_ref,
             lw_ref, lb_ref, h_out, hh_out, cc_out):
        agg = jnp.concatenate([acc_ref[0][:, :w2], acc_ref[1][:, :w2]],
                              axis=1)
        degc = acc_ref[0][:, w2:w2 + 1]
        scale = 1.0 / jnp.maximum(degc, 1.0)
        aggx = agg[:, :d_in] * scale
        aggh = agg[:, d_in:wx] * scale
        z = jnp.dot(aggx, wf_ref[0:d_in, :],
                    preferred_element_type=jnp.float32)
        z += jnp.dot(x_ref[...], wf_ref[d_in:2 * d_in, :],
                     preferred_element_type=jnp.float32)
        z += jnp.dot(aggh, wf_ref[2 * d_in:2 * d_in + d_out, :],
                     preferred_element_type=jnp.float32)
        z += jnp.dot(h_ref[...], wf_ref[2 * d_in + d_out:, :],
                     preferred_element_type=jnp.float32)
        z += b_ref[...]
        ig = jax.nn.sigmoid(z[:, 0:d_out])
        fg = jax.nn.sigmoid(z[:, d_out:2 * d_out])
        tg = jnp.tanh(z[:, 2 * d_out:3 * d_out])
        og = jax.nn.sigmoid(z[:, 3 * d_out:4 * d_out])
        cc = fg * c_ref[...] + ig * tg
        hh = og * jnp.tanh(cc)
        hr = jnp.maximum(hh, 0.0)
        h_out[...] = jnp.dot(hr, lw_ref[...],
                             preferred_element_type=jnp.float32) + lb_ref[...]
        hh_out[...] = hh
        cc_out[...] = cc

    grid = (n // blk,)
    k_total = 2 * d_in + 2 * d_out
    return pl.pallas_call(
        body,
        grid=grid,
        in_specs=[
            pl.BlockSpec((_NC, blk, wr), lambda i: (0, i, 0)),
            pl.BlockSpec((blk, d_in), lambda i: (i, 0)),
            pl.BlockSpec((blk, d_out), lambda i: (i, 0)),
            pl.BlockSpec((blk, d_out), lambda i: (i, 0)),
            pl.BlockSpec((k_total, 4 * d_out), lambda i: (0, 0)),
            pl.BlockSpec((1, 4 * d_out), lambda i: (0, 0)),
            pl.BlockSpec((d_out, periods), lambda i: (0, 0)),
            pl.BlockSpec((1, periods), lambda i: (0, 0)),
        ],
        out_specs=[
            pl.BlockSpec((blk, periods), lambda i: (i, 0)),
            pl.BlockSpec((blk, d_out), lambda i: (i, 0)),
            pl.BlockSpec((blk, d_out), lambda i: (i, 0)),
        ],
        out_shape=[
            jax.ShapeDtypeStruct((n, periods), jnp.float32),
            jax.ShapeDtypeStruct((n, d_out), jnp.float32),
            jax.ShapeDtypeStruct((n, d_out), jnp.float32),
        ],
    )(acc, x, h_0, c_0, w_full, bias, lin_w, lin_b)


def kernel(x, edge_index, edge_weight, h_0, c_0, params):
    del edge_weight
    n, d_in = x.shape
    d_out = h_0.shape[1]
    periods = params["lin_W"].shape[1]

    n_pad = ((n + _NS * _L - 1) // (_NS * _L)) * (_NS * _L)

    xh = jnp.concatenate([x, h_0], axis=1)
    w2 = xh.shape[1] // 2
    one = jnp.ones((n, _DEG_W), jnp.float32)
    xh_split = jnp.stack(
        [jnp.concatenate([xh[:, :w2], one], axis=1),
         jnp.concatenate([xh[:, w2:], one], axis=1)], axis=0)
    src = edge_index[0]
    dst = edge_index[1]

    acc = _sc_aggregate(xh_split, src, dst, n_pad)

    wxg = jnp.concatenate([params["x_%s_W" % g] for g in "ifco"], axis=1)
    rxg = jnp.concatenate([params["x_%s_root" % g] for g in "ifco"], axis=1)
    whg = jnp.concatenate([params["h_%s_W" % g] for g in "ifco"], axis=1)
    rhg = jnp.concatenate([params["h_%s_root" % g] for g in "ifco"], axis=1)
    w_full = jnp.concatenate([wxg, rxg, whg, rhg], axis=0)
    bias = jnp.concatenate(
        [params["x_%s_bias" % g] + params["h_%s_bias" % g] for g in "ifco"]
    ).reshape(1, 4 * d_out)
    lin_b = params["lin_b"].reshape(1, periods)

    h, hh, cc = _tc_dense(acc, x, h_0, c_0, w_full, bias,
                          params["lin_W"], lin_b, d_in, d_out, periods,
                          blk=2000)
    return (h, hh, cc)


# speedup vs baseline: 1.0678x; 1.0678x over previous
import functools

import jax
import jax.numpy as jnp
from jax import lax
from jax.experimental import pallas as pl
from jax.experimental.pallas import tpu as pltpu
from jax.experimental.pallas import tpu_sc as plsc

_NC = 2
_NS = 16
_L = 16
_NW = _NC * _NS

_CHUNK = 80
_K = 5
_DEG_W = 16


def _sc_aggregate(xh_split, src, dst, n_pad):
    e_total = src.shape[0]
    w = xh_split.shape[2]
    assert e_total % (_NS * _CHUNK * _K) == 0
    e_per_t = e_total // _NS
    n_groups = e_per_t // (_CHUNK * _K)
    rows_per_tile = n_pad // _NS

    mesh = plsc.VectorSubcoreMesh(
        core_axis_name="c", subcore_axis_name="s",
        num_cores=_NC, num_subcores=_NS)

    @functools.partial(
        pl.kernel,
        out_type=(
            jax.ShapeDtypeStruct((_NC, n_pad, w), jnp.float32),
            jax.ShapeDtypeStruct((_NC, n_pad, _DEG_W), jnp.float32),
        ),
        mesh=mesh,
        scratch_types=[
            pltpu.VMEM((2, _K, _CHUNK), jnp.int32),
            pltpu.VMEM((2, _K, _CHUNK), jnp.int32),
            pltpu.VMEM((2, _K, _CHUNK, w), jnp.float32),
            pltpu.VMEM((_CHUNK, _DEG_W), jnp.float32),
            pltpu.VMEM((_L, w), jnp.float32),
            pltpu.VMEM((_L, _DEG_W), jnp.float32),
            pltpu.VMEM_SHARED((n_pad, w), jnp.float32),
            pltpu.VMEM_SHARED((n_pad, _DEG_W), jnp.float32),
            pltpu.SemaphoreType.DMA,
            pltpu.SemaphoreType.DMA,
            pltpu.SemaphoreType.DMA,
            pltpu.SemaphoreType.DMA,
            pltpu.SemaphoreType.DMA,
            pltpu.SemaphoreType.DMA,
            pltpu.SemaphoreType.DMA,
            pltpu.SemaphoreType.DMA,
        ],
        compiler_params=pltpu.CompilerParams(use_tc_tiling_on_sc=False),
    )
    def agg_kernel(xh_hbm, src_hbm, dst_hbm, acc_out, deg_out,
                   srcv, dstv, rows, ones, zrow, zdeg, acc_sh, deg_sh,
                   sem_is0, sem_is1, sem_id0, sem_id1,
                   sem_g0, sem_g1, sem_s0, sem_s1):
        cid = lax.axis_index("c")
        sid = lax.axis_index("s")
        sem_is = (sem_is0, sem_is1)
        sem_id = (sem_id0, sem_id1)
        sem_g = (sem_g0, sem_g1)
        sem_s = (sem_s0, sem_s1)

        for i in range(_L):
            for k in range(w // _L):
                zrow[i, pl.ds(k * _L, _L)] = jnp.zeros((_L,), jnp.float32)
            zdeg[i, pl.ds(0, _L)] = jnp.zeros((_L,), jnp.float32)
        for i in range(_CHUNK):
            ones[i, pl.ds(0, _L)] = jnp.ones((_L,), jnp.float32)

        zbase = sid * rows_per_tile

        @pl.loop(0, rows_per_tile, step=_L)
        def _(r):
            pltpu.sync_copy(zrow, acc_sh.at[pl.ds(zbase + r, _L)])
            pltpu.sync_copy(zdeg, deg_sh.at[pl.ds(zbase + r, _L)])

        plsc.subcore_barrier()

        ebase = sid * e_per_t

        def issue_src_idx(g, p):
            for b in range(_K):
                off = pl.multiple_of(ebase + (g * _K + b) * _CHUNK, 8)
                pltpu.async_copy(src_hbm.at[pl.ds(off, _CHUNK)],
                                 srcv.at[p, b], sem_is[p])

        def issue_dst_idx(g, p):
            for b in range(_K):
                off = pl.multiple_of(ebase + (g * _K + b) * _CHUNK, 8)
                pltpu.async_copy(dst_hbm.at[pl.ds(off, _CHUNK)],
                                 dstv.at[p, b], sem_id[p])

        def drain_src_idx(p):
            for b in range(_K):
                pltpu.make_async_copy(src_hbm.at[pl.ds(ebase, _CHUNK)],
                                      srcv.at[p, b], sem_is[p]).wait()

        def drain_dst_idx(p):
            for b in range(_K):
                pltpu.make_async_copy(dst_hbm.at[pl.ds(ebase, _CHUNK)],
                                      dstv.at[p, b], sem_id[p]).wait()

        def drain_scatters(p):
            for b in range(_K):
                pltpu.make_async_copy(rows.at[p, b],
                                      acc_sh.at[dstv.at[p, b]],
                                      sem_s[p]).wait()
                pltpu.make_async_copy(ones, deg_sh.at[dstv.at[p, b]],
                                      sem_s[p]).wait()

        issue_src_idx(0, 0)
        issue_src_idx(1, 1)

        @pl.loop(0, n_groups, step=2)
        def _(g):
            for p in range(2):
                @pl.when(g >= 2)
                def _():
                    drain_scatters(p)
                issue_dst_idx(g + p, p)
                drain_src_idx(p)
                gathers = [
                    pltpu.async_copy(xh_hbm.at[cid].at[srcv.at[p, b]],
                                     rows.at[p, b], sem_g[p])
                    for b in range(_K)
                ]

                for d in gathers:
                    d.wait()

                @pl.when(g + p + 2 < n_groups)
                def _():
                    issue_src_idx(g + p + 2, p)
                drain_dst_idx(p)
                for b in range(_K):
                    pltpu.async_copy(rows.at[p, b],
                                     acc_sh.at[dstv.at[p, b]],
                                     sem_s[p], add=True)
                    pltpu.async_copy(ones, deg_sh.at[dstv.at[p, b]],
                                     sem_s[p], add=True)

        drain_scatters(0)
        drain_scatters(1)

        plsc.subcore_barrier()

        pltpu.sync_copy(acc_sh.at[pl.ds(zbase, rows_per_tile)],
                        acc_out.at[cid, pl.ds(zbase, rows_per_tile)])
        pltpu.sync_copy(deg_sh.at[pl.ds(zbase, rows_per_tile)],
                        deg_out.at[cid, pl.ds(zbase, rows_per_tile)])

    return agg_kernel(xh_split, src, dst)


def _tc_dense(acc, deg, x, h_0, c_0, w_full, bias, lin_w, lin_b,
              d_in, d_out, periods, blk):
    n = x.shape[0]
    assert n % blk == 0
    wx = d_in + d_out

    w2 = wx // 2

    def body(acc_ref, deg_ref, x_ref, h_ref, c_ref, wf_ref, b_ref,
             lw_ref, lb_ref, h_out, hh_out, cc_out):
        agg = jnp.concatenate([acc_ref[0], acc_ref[1]], axis=1)
        degc = deg_ref[0][:, 0:1]
        scale = 1.0 / jnp.maximum(degc, 1.0)
        aggx = agg[:, :d_in] * scale
        aggh = agg[:, d_in:wx] * scale
        z = jnp.dot(aggx, wf_ref[0:d_in, :],
                    preferred_element_type=jnp.float32)
        z += jnp.dot(x_ref[...], wf_ref[d_in:2 * d_in, :],
                     preferred_element_type=jnp.float32)
        z += jnp.dot(aggh, wf_ref[2 * d_in:2 * d_in + d_out, :],
                     preferred_element_type=jnp.float32)
        z += jnp.dot(h_ref[...], wf_ref[2 * d_in + d_out:, :],
                     preferred_element_type=jnp.float32)
        z += b_ref[...]
        ig = jax.nn.sigmoid(z[:, 0:d_out])
        fg = jax.nn.sigmoid(z[:, d_out:2 * d_out])
        tg = jnp.tanh(z[:, 2 * d_out:3 * d_out])
        og = jax.nn.sigmoid(z[:, 3 * d_out:4 * d_out])
        cc = fg * c_ref[...] + ig * tg
        hh = og * jnp.tanh(cc)
        hr = jnp.maximum(hh, 0.0)
        h_out[...] = jnp.dot(hr, lw_ref[...],
                             preferred_element_type=jnp.float32) + lb_ref[...]
        hh_out[...] = hh
        cc_out[...] = cc

    grid = (n // blk,)
    k_total = 2 * d_in + 2 * d_out
    return pl.pallas_call(
        body,
        grid=grid,
        in_specs=[
            pl.BlockSpec((_NC, blk, w2), lambda i: (0, i, 0)),
            pl.BlockSpec((1, blk, _DEG_W), lambda i: (0, i, 0)),
            pl.BlockSpec((blk, d_in), lambda i: (i, 0)),
            pl.BlockSpec((blk, d_out), lambda i: (i, 0)),
            pl.BlockSpec((blk, d_out), lambda i: (i, 0)),
            pl.BlockSpec((k_total, 4 * d_out), lambda i: (0, 0)),
            pl.BlockSpec((1, 4 * d_out), lambda i: (0, 0)),
            pl.BlockSpec((d_out, periods), lambda i: (0, 0)),
            pl.BlockSpec((1, periods), lambda i: (0, 0)),
        ],
        out_specs=[
            pl.BlockSpec((blk, periods), lambda i: (i, 0)),
            pl.BlockSpec((blk, d_out), lambda i: (i, 0)),
            pl.BlockSpec((blk, d_out), lambda i: (i, 0)),
        ],
        out_shape=[
            jax.ShapeDtypeStruct((n, periods), jnp.float32),
            jax.ShapeDtypeStruct((n, d_out), jnp.float32),
            jax.ShapeDtypeStruct((n, d_out), jnp.float32),
        ],
    )(acc, deg, x, h_0, c_0, w_full, bias, lin_w, lin_b)


def kernel(x, edge_index, edge_weight, h_0, c_0, params):
    del edge_weight
    n, d_in = x.shape
    d_out = h_0.shape[1]
    periods = params["lin_W"].shape[1]

    n_pad = ((n + _NS * _L - 1) // (_NS * _L)) * (_NS * _L)

    xh = jnp.concatenate([x, h_0], axis=1)
    w2 = xh.shape[1] // 2
    xh_split = jnp.stack([xh[:, :w2], xh[:, w2:]], axis=0)
    src = edge_index[0]
    dst = edge_index[1]

    acc, deg = _sc_aggregate(xh_split, src, dst, n_pad)

    wxg = jnp.concatenate([params["x_%s_W" % g] for g in "ifco"], axis=1)
    rxg = jnp.concatenate([params["x_%s_root" % g] for g in "ifco"], axis=1)
    whg = jnp.concatenate([params["h_%s_W" % g] for g in "ifco"], axis=1)
    rhg = jnp.concatenate([params["h_%s_root" % g] for g in "ifco"], axis=1)
    w_full = jnp.concatenate([wxg, rxg, whg, rhg], axis=0)
    bias = jnp.concatenate(
        [params["x_%s_bias" % g] + params["h_%s_bias" % g] for g in "ifco"]
    ).reshape(1, 4 * d_out)
    lin_b = params["lin_b"].reshape(1, periods)

    h, hh, cc = _tc_dense(acc, deg, x, h_0, c_0, w_full, bias,
                          params["lin_W"], lin_b, d_in, d_out, periods,
                          blk=10000)
    return (h, hh, cc)


# speedup vs baseline: 1.0759x; 1.0076x over previous
import functools

import jax
import jax.numpy as jnp
from jax import lax
from jax.experimental import pallas as pl
from jax.experimental.pallas import tpu as pltpu
from jax.experimental.pallas import tpu_sc as plsc

_NC = 2
_NS = 16
_L = 16
_NW = _NC * _NS

_CHUNK = 80
_K = 5
_DEG_W = 16


def _sc_aggregate(xh_split, src, dst, n_pad):
    e_total = src.shape[0]
    w = xh_split.shape[2]
    assert e_total % (_NS * _CHUNK * _K) == 0
    e_per_t = e_total // _NS
    n_groups = e_per_t // (_CHUNK * _K)
    rows_per_tile = n_pad // _NS

    mesh = plsc.VectorSubcoreMesh(
        core_axis_name="c", subcore_axis_name="s",
        num_cores=_NC, num_subcores=_NS)

    @functools.partial(
        pl.kernel,
        out_type=(
            jax.ShapeDtypeStruct((_NC, n_pad, w), jnp.float32),
            jax.ShapeDtypeStruct((_NC, n_pad, _DEG_W), jnp.float32),
        ),
        mesh=mesh,
        scratch_types=[
            pltpu.VMEM((2, _K, _CHUNK), jnp.int32),
            pltpu.VMEM((2, _K, _CHUNK), jnp.int32),
            pltpu.VMEM((2, _K, _CHUNK, w), jnp.float32),
            pltpu.VMEM((_CHUNK, _DEG_W), jnp.float32),
            pltpu.VMEM((_L, w), jnp.float32),
            pltpu.VMEM((_L, _DEG_W), jnp.float32),
            pltpu.VMEM_SHARED((n_pad, w), jnp.float32),
            pltpu.VMEM_SHARED((n_pad, _DEG_W), jnp.float32),
            pltpu.SemaphoreType.DMA,
            pltpu.SemaphoreType.DMA,
            pltpu.SemaphoreType.DMA,
            pltpu.SemaphoreType.DMA,
            pltpu.SemaphoreType.DMA,
            pltpu.SemaphoreType.DMA,
            pltpu.SemaphoreType.DMA,
            pltpu.SemaphoreType.DMA,
        ],
        compiler_params=pltpu.CompilerParams(use_tc_tiling_on_sc=False),
    )
    def agg_kernel(xh_hbm, src_hbm, dst_hbm, acc_out, deg_out,
                   srcv, dstv, rows, ones, zrow, zdeg, acc_sh, deg_sh,
                   sem_is0, sem_is1, sem_id0, sem_id1,
                   sem_g0, sem_g1, sem_s0, sem_s1):
        cid = lax.axis_index("c")
        sid = lax.axis_index("s")
        sem_is = (sem_is0, sem_is1)
        sem_id = (sem_id0, sem_id1)
        sem_g = (sem_g0, sem_g1)
        sem_s = (sem_s0, sem_s1)

        for i in range(_L):
            for k in range(w // _L):
                zrow[i, pl.ds(k * _L, _L)] = jnp.zeros((_L,), jnp.float32)
            zdeg[i, pl.ds(0, _L)] = jnp.zeros((_L,), jnp.float32)
        for i in range(_CHUNK):
            ones[i, pl.ds(0, _L)] = jnp.ones((_L,), jnp.float32)

        zbase = sid * rows_per_tile

        @pl.loop(0, rows_per_tile, step=_L)
        def _(r):
            pltpu.sync_copy(zrow, acc_sh.at[pl.ds(zbase + r, _L)])
            pltpu.sync_copy(zdeg, deg_sh.at[pl.ds(zbase + r, _L)])

        plsc.subcore_barrier()

        ebase = sid * e_per_t

        def issue_src_idx(g, p):
            for b in range(_K):
                off = pl.multiple_of(ebase + (g * _K + b) * _CHUNK, 8)
                pltpu.async_copy(src_hbm.at[pl.ds(off, _CHUNK)],
                                 srcv.at[p, b], sem_is[p])

        def issue_dst_idx(g, p):
            for b in range(_K):
                off = pl.multiple_of(ebase + (g * _K + b) * _CHUNK, 8)
                pltpu.async_copy(dst_hbm.at[pl.ds(off, _CHUNK)],
                                 dstv.at[p, b], sem_id[p])

        def drain_src_idx(p):
            for b in range(_K):
                pltpu.make_async_copy(src_hbm.at[pl.ds(ebase, _CHUNK)],
                                      srcv.at[p, b], sem_is[p]).wait()

        def drain_dst_idx(p):
            for b in range(_K):
                pltpu.make_async_copy(dst_hbm.at[pl.ds(ebase, _CHUNK)],
                                      dstv.at[p, b], sem_id[p]).wait()

        def drain_scatters(p):
            for b in range(_K):
                pltpu.make_async_copy(rows.at[p, b],
                                      acc_sh.at[dstv.at[p, b]],
                                      sem_s[p]).wait()
                pltpu.make_async_copy(ones, deg_sh.at[dstv.at[p, b]],
                                      sem_s[p]).wait()

        issue_src_idx(0, 0)
        issue_src_idx(1, 1)

        @pl.loop(0, n_groups, step=2)
        def _(g):
            for p in range(2):
                @pl.when(g >= 2)
                def _():
                    drain_scatters(p)
                issue_dst_idx(g + p, p)
                drain_src_idx(p)
                gathers = [
                    pltpu.async_copy(xh_hbm.at[cid].at[srcv.at[p, b]],
                                     rows.at[p, b], sem_g[p])
                    for b in range(_K)
                ]

                for d in gathers:
                    d.wait()

                @pl.when(g + p + 2 < n_groups)
                def _():
                    issue_src_idx(g + p + 2, p)
                drain_dst_idx(p)
                for b in range(_K):
                    pltpu.async_copy(rows.at[p, b],
                                     acc_sh.at[dstv.at[p, b]],
                                     sem_s[p], add=True)
                    pltpu.async_copy(ones, deg_sh.at[dstv.at[p, b]],
                                     sem_s[p], add=True)

        drain_scatters(0)
        drain_scatters(1)

        plsc.subcore_barrier()

        pltpu.sync_copy(acc_sh.at[pl.ds(zbase, rows_per_tile)],
                        acc_out.at[cid, pl.ds(zbase, rows_per_tile)])
        pltpu.sync_copy(deg_sh.at[pl.ds(zbase, rows_per_tile)],
                        deg_out.at[cid, pl.ds(zbase, rows_per_tile)])

    return agg_kernel(xh_split, src, dst)


def _tc_dense(acc, deg, x, h_0, c_0, w_full, bias, lin_w, lin_b,
              d_in, d_out, periods, blk):
    n = x.shape[0]
    assert n % blk == 0
    wx = d_in + d_out

    w2 = wx // 2

    def body(acc_ref, deg_ref, x_ref, h_ref, c_ref, wf_ref, b_ref,
             lw_ref, lb_ref, h_out, hh_out, cc_out):
        agg = jnp.concatenate([acc_ref[0], acc_ref[1]], axis=1)
        degc = deg_ref[0][:, 0:1]
        scale = 1.0 / jnp.maximum(degc, 1.0)
        aggx = agg[:, :d_in] * scale
        aggh = agg[:, d_in:wx] * scale
        z = jnp.dot(aggx, wf_ref[0:d_in, :],
                    preferred_element_type=jnp.float32)
        z += jnp.dot(x_ref[...], wf_ref[d_in:2 * d_in, :],
                     preferred_element_type=jnp.float32)
        z += jnp.dot(aggh, wf_ref[2 * d_in:2 * d_in + d_out, :],
                     preferred_element_type=jnp.float32)
        z += jnp.dot(h_ref[...], wf_ref[2 * d_in + d_out:, :],
                     preferred_element_type=jnp.float32)
        z += b_ref[...]
        ig = jax.nn.sigmoid(z[:, 0:d_out])
        fg = jax.nn.sigmoid(z[:, d_out:2 * d_out])
        tg = jnp.tanh(z[:, 2 * d_out:3 * d_out])
        og = jax.nn.sigmoid(z[:, 3 * d_out:4 * d_out])
        cc = fg * c_ref[...] + ig * tg
        hh = og * jnp.tanh(cc)
        hr = jnp.maximum(hh, 0.0)
        h_out[...] = jnp.dot(hr, lw_ref[...],
                             preferred_element_type=jnp.float32) + lb_ref[...]
        hh_out[...] = hh
        cc_out[...] = cc

    grid = (n // blk,)
    k_total = 2 * d_in + 2 * d_out
    return pl.pallas_call(
        body,
        grid=grid,
        in_specs=[
            pl.BlockSpec((_NC, blk, w2), lambda i: (0, i, 0)),
            pl.BlockSpec((1, blk, _DEG_W), lambda i: (0, i, 0)),
            pl.BlockSpec((blk, d_in), lambda i: (i, 0)),
            pl.BlockSpec((blk, d_out), lambda i: (i, 0)),
            pl.BlockSpec((blk, d_out), lambda i: (i, 0)),
            pl.BlockSpec((k_total, 4 * d_out), lambda i: (0, 0)),
            pl.BlockSpec((1, 4 * d_out), lambda i: (0, 0)),
            pl.BlockSpec((d_out, periods), lambda i: (0, 0)),
            pl.BlockSpec((1, periods), lambda i: (0, 0)),
        ],
        out_specs=[
            pl.BlockSpec((blk, periods), lambda i: (i, 0)),
            pl.BlockSpec((blk, d_out), lambda i: (i, 0)),
            pl.BlockSpec((blk, d_out), lambda i: (i, 0)),
        ],
        out_shape=[
            jax.ShapeDtypeStruct((n, periods), jnp.float32),
            jax.ShapeDtypeStruct((n, d_out), jnp.float32),
            jax.ShapeDtypeStruct((n, d_out), jnp.float32),
        ],
    )(acc, deg, x, h_0, c_0, w_full, bias, lin_w, lin_b)


def kernel(x, edge_index, edge_weight, h_0, c_0, params):
    del edge_weight
    n, d_in = x.shape
    d_out = h_0.shape[1]
    periods = params["lin_W"].shape[1]

    n_pad = ((n + _NS * _L - 1) // (_NS * _L)) * (_NS * _L)

    xh = jnp.concatenate([x, h_0], axis=1)
    w2 = xh.shape[1] // 2
    xh_split = jnp.stack([xh[:, :w2], xh[:, w2:]], axis=0)
    src = edge_index[0]
    dst = edge_index[1]

    acc, deg = _sc_aggregate(xh_split, src, dst, n_pad)

    wxg = jnp.concatenate([params["x_%s_W" % g] for g in "ifco"], axis=1)
    rxg = jnp.concatenate([params["x_%s_root" % g] for g in "ifco"], axis=1)
    whg = jnp.concatenate([params["h_%s_W" % g] for g in "ifco"], axis=1)
    rhg = jnp.concatenate([params["h_%s_root" % g] for g in "ifco"], axis=1)
    w_full = jnp.concatenate([wxg, rxg, whg, rhg], axis=0)
    bias = jnp.concatenate(
        [params["x_%s_bias" % g] + params["h_%s_bias" % g] for g in "ifco"]
    ).reshape(1, 4 * d_out)
    lin_b = params["lin_b"].reshape(1, periods)

    h, hh, cc = _tc_dense(acc, deg, x, h_0, c_0, w_full, bias,
                          params["lin_W"], lin_b, d_in, d_out, periods,
                          blk=1000)
    return (h, hh, cc)


# speedup vs baseline: 1.0933x; 1.0161x over previous
import functools

import jax
import jax.numpy as jnp
from jax import lax
from jax.experimental import pallas as pl
from jax.experimental.pallas import tpu as pltpu
from jax.experimental.pallas import tpu_sc as plsc

_NC = 2
_NS = 16
_L = 16
_NW = _NC * _NS

_CHUNK = 80
_K = 5
_DEG_W = 16


def _sc_aggregate(xh_split, src, dst, n_pad):
    e_total = src.shape[0]
    w = xh_split.shape[2]
    assert e_total % (_NS * _CHUNK * _K) == 0
    e_per_t = e_total // _NS
    n_groups = e_per_t // (_CHUNK * _K)
    rows_per_tile = n_pad // _NS

    mesh = plsc.VectorSubcoreMesh(
        core_axis_name="c", subcore_axis_name="s",
        num_cores=_NC, num_subcores=_NS)

    @functools.partial(
        pl.kernel,
        out_type=(
            jax.ShapeDtypeStruct((_NC, n_pad, w), jnp.float32),
            jax.ShapeDtypeStruct((_NC, n_pad // _L, _L), jnp.float32),
        ),
        mesh=mesh,
        scratch_types=[
            pltpu.VMEM((2, _K, _CHUNK), jnp.int32),
            pltpu.VMEM((2, _K, _CHUNK), jnp.int32),
            pltpu.VMEM((2, _K, _CHUNK, w), jnp.float32),
            pltpu.VMEM((_L, w), jnp.float32),
            pltpu.VMEM((n_pad // _L, _L), jnp.float32),
            pltpu.VMEM((n_pad // _L,), jnp.int32),
            pltpu.VMEM_SHARED((n_pad, w), jnp.float32),
            pltpu.VMEM_SHARED((n_pad // _L, _L), jnp.float32),
            pltpu.SemaphoreType.DMA,
            pltpu.SemaphoreType.DMA,
            pltpu.SemaphoreType.DMA,
            pltpu.SemaphoreType.DMA,
            pltpu.SemaphoreType.DMA,
            pltpu.SemaphoreType.DMA,
            pltpu.SemaphoreType.DMA,
            pltpu.SemaphoreType.DMA,
        ],
        compiler_params=pltpu.CompilerParams(
            use_tc_tiling_on_sc=False, needs_layout_passes=False),
    )
    def agg_kernel(xh_hbm, src_hbm, dst_hbm, acc_out, deg_out,
                   srcv, dstv, rows, zrow, ldeg, idv, acc_sh, deg_sh,
                   sem_is0, sem_is1, sem_id0, sem_id1,
                   sem_g0, sem_g1, sem_s0, sem_s1):
        cid = lax.axis_index("c")
        sid = lax.axis_index("s")
        sem_is = (sem_is0, sem_is1)
        sem_id = (sem_id0, sem_id1)
        sem_g = (sem_g0, sem_g1)
        sem_s = (sem_s0, sem_s1)

        for i in range(_L):
            for k in range(w // _L):
                zrow[i, pl.ds(k * _L, _L)] = jnp.zeros((_L,), jnp.float32)
        deg_rows = n_pad // _L

        @pl.loop(0, deg_rows)
        def _(r):
            ldeg[r, pl.ds(0, _L)] = jnp.zeros((_L,), jnp.float32)

        @pl.loop(0, deg_rows, step=_L)
        def _(r):
            idv[pl.ds(r, _L)] = lax.iota(jnp.int32, _L) + r

        zbase = sid * rows_per_tile

        @pl.loop(0, rows_per_tile, step=_L)
        def _(r):
            pltpu.sync_copy(zrow, acc_sh.at[pl.ds(zbase + r, _L)])
        dpt = deg_rows // _NS
        pltpu.sync_copy(ldeg.at[pl.ds(0, dpt)],
                        deg_sh.at[pl.ds(sid * dpt, dpt)])

        plsc.subcore_barrier()

        ebase = sid * e_per_t

        def issue_src_idx(g, p):
            for b in range(_K):
                off = pl.multiple_of(ebase + (g * _K + b) * _CHUNK, 8)
                pltpu.async_copy(src_hbm.at[pl.ds(off, _CHUNK)],
                                 srcv.at[p, b], sem_is[p])

        def issue_dst_idx(g, p):
            for b in range(_K):
                off = pl.multiple_of(ebase + (g * _K + b) * _CHUNK, 8)
                pltpu.async_copy(dst_hbm.at[pl.ds(off, _CHUNK)],
                                 dstv.at[p, b], sem_id[p])

        def drain_src_idx(p):
            for b in range(_K):
                pltpu.make_async_copy(src_hbm.at[pl.ds(ebase, _CHUNK)],
                                      srcv.at[p, b], sem_is[p]).wait()

        def drain_dst_idx(p):
            for b in range(_K):
                pltpu.make_async_copy(dst_hbm.at[pl.ds(ebase, _CHUNK)],
                                      dstv.at[p, b], sem_id[p]).wait()

        def drain_scatters(p):
            for b in range(_K):
                pltpu.make_async_copy(rows.at[p, b],
                                      acc_sh.at[dstv.at[p, b]],
                                      sem_s[p]).wait()

        issue_src_idx(0, 0)
        issue_src_idx(1, 1)

        @pl.loop(0, n_groups, step=2)
        def _(g):
            for p in range(2):
                @pl.when(g >= 2)
                def _():
                    drain_scatters(p)
                issue_dst_idx(g + p, p)
                drain_src_idx(p)
                gathers = [
                    pltpu.async_copy(xh_hbm.at[cid].at[srcv.at[p, b]],
                                     rows.at[p, b], sem_g[p])
                    for b in range(_K)
                ]

                for d in gathers:
                    d.wait()

                @pl.when(g + p + 2 < n_groups)
                def _():
                    issue_src_idx(g + p + 2, p)
                drain_dst_idx(p)
                for b in range(_K):
                    pltpu.async_copy(rows.at[p, b],
                                     acc_sh.at[dstv.at[p, b]],
                                     sem_s[p], add=True)
                vone = jnp.ones((_L,), jnp.float32)
                for b in range(_K):
                    for k in range(_CHUNK // _L):
                        v = dstv[p, b, pl.ds(k * _L, _L)]
                        plsc.addupdate_scatter(
                            ldeg,
                            [lax.shift_right_logical(v, 4),
                             lax.bitwise_and(v, 15)],
                            vone)

        drain_scatters(0)
        drain_scatters(1)

        pltpu.sync_copy(ldeg, deg_sh.at[idv], add=True)

        plsc.subcore_barrier()

        pltpu.sync_copy(acc_sh.at[pl.ds(zbase, rows_per_tile)],
                        acc_out.at[cid, pl.ds(zbase, rows_per_tile)])
        pltpu.sync_copy(deg_sh.at[pl.ds(sid * dpt, dpt)],
                        deg_out.at[cid, pl.ds(sid * dpt, dpt)])

    return agg_kernel(xh_split, src, dst)


def _tc_dense(acc, deg, x, h_0, c_0, w_full, bias, lin_w, lin_b,
              d_in, d_out, periods, blk):
    n = x.shape[0]
    assert n % blk == 0
    wx = d_in + d_out

    w2 = wx // 2

    def body(acc_ref, deg_ref, x_ref, h_ref, c_ref, wf_ref, b_ref,
             lw_ref, lb_ref, h_out, hh_out, cc_out):
        agg = jnp.concatenate([acc_ref[0], acc_ref[1]], axis=1)
        degc = deg_ref[...]
        scale = 1.0 / jnp.maximum(degc, 1.0)
        aggx = agg[:, :d_in] * scale
        aggh = agg[:, d_in:wx] * scale
        z = jnp.dot(aggx, wf_ref[0:d_in, :],
                    preferred_element_type=jnp.float32)
        z += jnp.dot(x_ref[...], wf_ref[d_in:2 * d_in, :],
                     preferred_element_type=jnp.float32)
        z += jnp.dot(aggh, wf_ref[2 * d_in:2 * d_in + d_out, :],
                     preferred_element_type=jnp.float32)
        z += jnp.dot(h_ref[...], wf_ref[2 * d_in + d_out:, :],
                     preferred_element_type=jnp.float32)
        z += b_ref[...]
        ig = jax.nn.sigmoid(z[:, 0:d_out])
        fg = jax.nn.sigmoid(z[:, d_out:2 * d_out])
        tg = jnp.tanh(z[:, 2 * d_out:3 * d_out])
        og = jax.nn.sigmoid(z[:, 3 * d_out:4 * d_out])
        cc = fg * c_ref[...] + ig * tg
        hh = og * jnp.tanh(cc)
        hr = jnp.maximum(hh, 0.0)
        h_out[...] = jnp.dot(hr, lw_ref[...],
                             preferred_element_type=jnp.float32) + lb_ref[...]
        hh_out[...] = hh
        cc_out[...] = cc

    grid = (n // blk,)
    k_total = 2 * d_in + 2 * d_out
    return pl.pallas_call(
        body,
        grid=grid,
        in_specs=[
            pl.BlockSpec((_NC, blk, w2), lambda i: (0, i, 0)),
            pl.BlockSpec((blk, 1), lambda i: (i, 0)),
            pl.BlockSpec((blk, d_in), lambda i: (i, 0)),
            pl.BlockSpec((blk, d_out), lambda i: (i, 0)),
            pl.BlockSpec((blk, d_out), lambda i: (i, 0)),
            pl.BlockSpec((k_total, 4 * d_out), lambda i: (0, 0)),
            pl.BlockSpec((1, 4 * d_out), lambda i: (0, 0)),
            pl.BlockSpec((d_out, periods), lambda i: (0, 0)),
            pl.BlockSpec((1, periods), lambda i: (0, 0)),
        ],
        out_specs=[
            pl.BlockSpec((blk, periods), lambda i: (i, 0)),
            pl.BlockSpec((blk, d_out), lambda i: (i, 0)),
            pl.BlockSpec((blk, d_out), lambda i: (i, 0)),
        ],
        out_shape=[
            jax.ShapeDtypeStruct((n, periods), jnp.float32),
            jax.ShapeDtypeStruct((n, d_out), jnp.float32),
            jax.ShapeDtypeStruct((n, d_out), jnp.float32),
        ],
    )(acc, deg, x, h_0, c_0, w_full, bias, lin_w, lin_b)


def kernel(x, edge_index, edge_weight, h_0, c_0, params):
    del edge_weight
    n, d_in = x.shape
    d_out = h_0.shape[1]
    periods = params["lin_W"].shape[1]

    n_pad = ((n + _NS * _L - 1) // (_NS * _L)) * (_NS * _L)

    xh = jnp.concatenate([x, h_0], axis=1)
    w2 = xh.shape[1] // 2
    xh_split = jnp.stack([xh[:, :w2], xh[:, w2:]], axis=0)
    src = edge_index[0]
    dst = edge_index[1]

    acc, deg = _sc_aggregate(xh_split, src, dst, n_pad)
    deg_col = deg[0].reshape(n_pad, 1)

    wxg = jnp.concatenate([params["x_%s_W" % g] for g in "ifco"], axis=1)
    rxg = jnp.concatenate([params["x_%s_root" % g] for g in "ifco"], axis=1)
    whg = jnp.concatenate([params["h_%s_W" % g] for g in "ifco"], axis=1)
    rhg = jnp.concatenate([params["h_%s_root" % g] for g in "ifco"], axis=1)
    w_full = jnp.concatenate([wxg, rxg, whg, rhg], axis=0)
    bias = jnp.concatenate(
        [params["x_%s_bias" % g] + params["h_%s_bias" % g] for g in "ifco"]
    ).reshape(1, 4 * d_out)
    lin_b = params["lin_b"].reshape(1, periods)

    h, hh, cc = _tc_dense(acc, deg_col, x, h_0, c_0, w_full, bias,
                          params["lin_W"], lin_b, d_in, d_out, periods,
                          blk=2000)
    return (h, hh, cc)


# speedup vs baseline: 1.1029x; 1.0088x over previous
import functools

import jax
import jax.numpy as jnp
from jax import lax
from jax.experimental import pallas as pl
from jax.experimental.pallas import tpu as pltpu
from jax.experimental.pallas import tpu_sc as plsc

_NC = 2
_NS = 16
_L = 16
_NW = _NC * _NS

_CHUNK = 80
_K = 5
_DEG_W = 16


def _sc_aggregate(xh_split, src, dst, n_pad):
    e_total = src.shape[0]
    w = xh_split.shape[2]
    assert e_total % (_NS * _CHUNK * _K) == 0
    e_per_t = e_total // _NS
    n_groups = e_per_t // (_CHUNK * _K)
    rows_per_tile = n_pad // _NS

    mesh = plsc.VectorSubcoreMesh(
        core_axis_name="c", subcore_axis_name="s",
        num_cores=_NC, num_subcores=_NS)

    @functools.partial(
        pl.kernel,
        out_type=(
            jax.ShapeDtypeStruct((_NC, n_pad, w), jnp.float32),
            jax.ShapeDtypeStruct((_NC, n_pad // _L, _L), jnp.float32),
        ),
        mesh=mesh,
        scratch_types=[
            pltpu.VMEM((2, _K, _CHUNK), jnp.int32),
            pltpu.VMEM((2, _K, _CHUNK), jnp.int32),
            pltpu.VMEM((2, _K, _CHUNK, w), jnp.float32),
            pltpu.VMEM((_L, w), jnp.float32),
            pltpu.VMEM((n_pad // _L, _L), jnp.float32),
            pltpu.VMEM((n_pad // _L,), jnp.int32),
            pltpu.VMEM_SHARED((n_pad, w), jnp.float32),
            pltpu.VMEM_SHARED((n_pad // _L, _L), jnp.float32),
            pltpu.SemaphoreType.DMA,
            pltpu.SemaphoreType.DMA,
            pltpu.SemaphoreType.DMA,
            pltpu.SemaphoreType.DMA,
            pltpu.SemaphoreType.DMA,
            pltpu.SemaphoreType.DMA,
            pltpu.SemaphoreType.DMA,
            pltpu.SemaphoreType.DMA,
        ],
        compiler_params=pltpu.CompilerParams(
            use_tc_tiling_on_sc=False, needs_layout_passes=False),
    )
    def agg_kernel(xh_hbm, src_hbm, dst_hbm, acc_out, deg_out,
                   srcv, dstv, rows, zrow, ldeg, idv, acc_sh, deg_sh,
                   sem_is0, sem_is1, sem_id0, sem_id1,
                   sem_g0, sem_g1, sem_s0, sem_s1):
        cid = lax.axis_index("c")
        sid = lax.axis_index("s")
        sem_is = (sem_is0, sem_is1)
        sem_id = (sem_id0, sem_id1)
        sem_g = (sem_g0, sem_g1)
        sem_s = (sem_s0, sem_s1)

        for i in range(_L):
            for k in range(w // _L):
                zrow[i, pl.ds(k * _L, _L)] = jnp.zeros((_L,), jnp.float32)
        deg_rows = n_pad // _L

        @pl.loop(0, deg_rows)
        def _(r):
            ldeg[r, pl.ds(0, _L)] = jnp.zeros((_L,), jnp.float32)

        @pl.loop(0, deg_rows, step=_L)
        def _(r):
            idv[pl.ds(r, _L)] = lax.iota(jnp.int32, _L) + r

        zbase = sid * rows_per_tile

        @pl.loop(0, rows_per_tile, step=_L)
        def _(r):
            pltpu.async_copy(zrow, acc_sh.at[pl.ds(zbase + r, _L)],
                             sem_g0)
        dpt = deg_rows // _NS
        pltpu.sync_copy(ldeg.at[pl.ds(0, dpt)],
                        deg_sh.at[pl.ds(sid * dpt, dpt)])
        for _z in range(rows_per_tile // _L):
            pltpu.make_async_copy(zrow, acc_sh.at[pl.ds(zbase, _L)],
                                  sem_g0).wait()

        plsc.subcore_barrier()

        ebase = sid * e_per_t

        def issue_src_idx(g, p):
            for b in range(_K):
                off = pl.multiple_of(ebase + (g * _K + b) * _CHUNK, 8)
                pltpu.async_copy(src_hbm.at[pl.ds(off, _CHUNK)],
                                 srcv.at[p, b], sem_is[p])

        def issue_dst_idx(g, p):
            for b in range(_K):
                off = pl.multiple_of(ebase + (g * _K + b) * _CHUNK, 8)
                pltpu.async_copy(dst_hbm.at[pl.ds(off, _CHUNK)],
                                 dstv.at[p, b], sem_id[p])

        def drain_src_idx(p):
            for b in range(_K):
                pltpu.make_async_copy(src_hbm.at[pl.ds(ebase, _CHUNK)],
                                      srcv.at[p, b], sem_is[p]).wait()

        def drain_dst_idx(p):
            for b in range(_K):
                pltpu.make_async_copy(dst_hbm.at[pl.ds(ebase, _CHUNK)],
                                      dstv.at[p, b], sem_id[p]).wait()

        def drain_scatters(p):
            for b in range(_K):
                pltpu.make_async_copy(rows.at[p, b],
                                      acc_sh.at[dstv.at[p, b]],
                                      sem_s[p]).wait()

        issue_src_idx(0, 0)
        issue_src_idx(1, 1)

        @pl.loop(0, n_groups, step=2)
        def _(g):
            for p in range(2):
                @pl.when(g >= 2)
                def _():
                    drain_scatters(p)
                issue_dst_idx(g + p, p)
                drain_src_idx(p)
                gathers = [
                    pltpu.async_copy(xh_hbm.at[cid].at[srcv.at[p, b]],
                                     rows.at[p, b], sem_g[p])
                    for b in range(_K)
                ]

                for d in gathers:
                    d.wait()

                @pl.when(g + p + 2 < n_groups)
                def _():
                    issue_src_idx(g + p + 2, p)
                drain_dst_idx(p)
                for b in range(_K):
                    pltpu.async_copy(rows.at[p, b],
                                     acc_sh.at[dstv.at[p, b]],
                                     sem_s[p], add=True)
                vone = jnp.ones((_L,), jnp.float32)
                for b in range(_K):
                    for k in range(_CHUNK // _L):
                        v = dstv[p, b, pl.ds(k * _L, _L)]
                        plsc.addupdate_scatter(
                            ldeg,
                            [lax.shift_right_logical(v, 4),
                             lax.bitwise_and(v, 15)],
                            vone)

        drain_scatters(0)
        drain_scatters(1)

        pltpu.sync_copy(ldeg, deg_sh.at[idv], add=True)

        plsc.subcore_barrier()

        pltpu.sync_copy(acc_sh.at[pl.ds(zbase, rows_per_tile)],
                        acc_out.at[cid, pl.ds(zbase, rows_per_tile)])
        pltpu.sync_copy(deg_sh.at[pl.ds(sid * dpt, dpt)],
                        deg_out.at[cid, pl.ds(sid * dpt, dpt)])

    return agg_kernel(xh_split, src, dst)


def _tc_dense(acc, deg, x, h_0, c_0, w_full, bias, lin_w, lin_b,
              d_in, d_out, periods, blk):
    n = x.shape[0]
    assert n % blk == 0
    wx = d_in + d_out

    w2 = wx // 2

    def body(acc_ref, deg_ref, x_ref, h_ref, c_ref, wf_ref, b_ref,
             lw_ref, lb_ref, h_out, hh_out, cc_out):
        agg = jnp.concatenate([acc_ref[0], acc_ref[1]], axis=1)
        degc = deg_ref[...]
        scale = 1.0 / jnp.maximum(degc, 1.0)
        aggx = agg[:, :d_in] * scale
        aggh = agg[:, d_in:wx] * scale
        z = jnp.dot(aggx, wf_ref[0:d_in, :],
                    preferred_element_type=jnp.float32)
        z += jnp.dot(x_ref[...], wf_ref[d_in:2 * d_in, :],
                     preferred_element_type=jnp.float32)
        z += jnp.dot(aggh, wf_ref[2 * d_in:2 * d_in + d_out, :],
                     preferred_element_type=jnp.float32)
        z += jnp.dot(h_ref[...], wf_ref[2 * d_in + d_out:, :],
                     preferred_element_type=jnp.float32)
        z += b_ref[...]
        ig = jax.nn.sigmoid(z[:, 0:d_out])
        fg = jax.nn.sigmoid(z[:, d_out:2 * d_out])
        tg = jnp.tanh(z[:, 2 * d_out:3 * d_out])
        og = jax.nn.sigmoid(z[:, 3 * d_out:4 * d_out])
        cc = fg * c_ref[...] + ig * tg
        hh = og * jnp.tanh(cc)
        hr = jnp.maximum(hh, 0.0)
        h_out[...] = jnp.dot(hr, lw_ref[...],
                             preferred_element_type=jnp.float32) + lb_ref[...]
        hh_out[...] = hh
        cc_out[...] = cc

    grid = (n // blk,)
    k_total = 2 * d_in + 2 * d_out
    return pl.pallas_call(
        body,
        grid=grid,
        in_specs=[
            pl.BlockSpec((_NC, blk, w2), lambda i: (0, i, 0)),
            pl.BlockSpec((blk, 1), lambda i: (i, 0)),
            pl.BlockSpec((blk, d_in), lambda i: (i, 0)),
            pl.BlockSpec((blk, d_out), lambda i: (i, 0)),
            pl.BlockSpec((blk, d_out), lambda i: (i, 0)),
            pl.BlockSpec((k_total, 4 * d_out), lambda i: (0, 0)),
            pl.BlockSpec((1, 4 * d_out), lambda i: (0, 0)),
            pl.BlockSpec((d_out, periods), lambda i: (0, 0)),
            pl.BlockSpec((1, periods), lambda i: (0, 0)),
        ],
        out_specs=[
            pl.BlockSpec((blk, periods), lambda i: (i, 0)),
            pl.BlockSpec((blk, d_out), lambda i: (i, 0)),
            pl.BlockSpec((blk, d_out), lambda i: (i, 0)),
        ],
        out_shape=[
            jax.ShapeDtypeStruct((n, periods), jnp.float32),
            jax.ShapeDtypeStruct((n, d_out), jnp.float32),
            jax.ShapeDtypeStruct((n, d_out), jnp.float32),
        ],
    )(acc, deg, x, h_0, c_0, w_full, bias, lin_w, lin_b)


def kernel(x, edge_index, edge_weight, h_0, c_0, params):
    del edge_weight
    n, d_in = x.shape
    d_out = h_0.shape[1]
    periods = params["lin_W"].shape[1]

    n_pad = ((n + _NS * _L - 1) // (_NS * _L)) * (_NS * _L)

    xh = jnp.concatenate([x, h_0], axis=1)
    w2 = xh.shape[1] // 2
    xh_split = jnp.stack([xh[:, :w2], xh[:, w2:]], axis=0)
    src = edge_index[0]
    dst = edge_index[1]

    acc, deg = _sc_aggregate(xh_split, src, dst, n_pad)
    deg_col = deg[0].reshape(n_pad, 1)

    wxg = jnp.concatenate([params["x_%s_W" % g] for g in "ifco"], axis=1)
    rxg = jnp.concatenate([params["x_%s_root" % g] for g in "ifco"], axis=1)
    whg = jnp.concatenate([params["h_%s_W" % g] for g in "ifco"], axis=1)
    rhg = jnp.concatenate([params["h_%s_root" % g] for g in "ifco"], axis=1)
    w_full = jnp.concatenate([wxg, rxg, whg, rhg], axis=0)
    bias = jnp.concatenate(
        [params["x_%s_bias" % g] + params["h_%s_bias" % g] for g in "ifco"]
    ).reshape(1, 4 * d_out)
    lin_b = params["lin_b"].reshape(1, periods)

    h, hh, cc = _tc_dense(acc, deg_col, x, h_0, c_0, w_full, bias,
                          params["lin_W"], lin_b, d_in, d_out, periods,
                          blk=2000)
    return (h, hh, cc)
